# Initial kernel scaffold; baseline (speedup 1.0000x reference)
import functools

import jax
import jax.numpy as jnp
from jax import lax
from jax.experimental import pallas as pl
from jax.experimental.pallas import tpu as pltpu
from jax.experimental.pallas import tpu_sc as plsc

N = 10000
E = 320000
D = 128

NC = 2
NS = 16
NW = NC * NS
CHUNK = 80
E_PER_W = E // NW
N_CHUNKS = E_PER_W // CHUNK
ROWS_PER_TILE = N // NS
ZB = 25
CNT_W = 16


def _mm_xt(a, w):
    return lax.dot_general(a, w, (((1,), (1,)), ((), ())),
                           preferred_element_type=jnp.float32)


def _tc1_body(x_ref, wl_ref, wr_ref, b_ref, y_ref, z_ref):
    xb = x_ref[...]
    y_ref[...] = _mm_xt(xb, wl_ref[...])
    z_ref[...] = _mm_xt(xb, wr_ref[...]) + b_ref[...]


def _tc2_body(acc_ref, cnt_ref, z_ref, wl_ref, wr_ref, b_ref, y_ref, z2_ref):
    a = acc_ref[0, :, :] + acc_ref[1, :, :]
    c = cnt_ref[0, :, 0:1] + cnt_ref[1, :, 0:1]
    inv = 1.0 / jnp.maximum(c, 1.0)
    h = jnp.maximum(a * inv + z_ref[...], 0.0)
    y_ref[...] = _mm_xt(h, wl_ref[...])
    z2_ref[...] = _mm_xt(h, wr_ref[...]) + b_ref[...]


def _tc3_body(acc_ref, cnt_ref, z_ref, out_ref):
    a = acc_ref[0, :, :] + acc_ref[1, :, :]
    c = cnt_ref[0, :, 0:1] + cnt_ref[1, :, 0:1]
    inv = 1.0 / jnp.maximum(c, 1.0)
    out_ref[...] = a * inv + z_ref[...]


_BM = 1000


def _tc1(x, wl, wr, b):
    grid = (N // _BM,)
    return pl.pallas_call(
        _tc1_body,
        grid=grid,
        in_specs=[
            pl.BlockSpec((_BM, D), lambda i: (i, 0)),
            pl.BlockSpec((D, D), lambda i: (0, 0)),
            pl.BlockSpec((D, D), lambda i: (0, 0)),
            pl.BlockSpec((1, D), lambda i: (0, 0)),
        ],
        out_specs=[
            pl.BlockSpec((_BM, D), lambda i: (i, 0)),
            pl.BlockSpec((_BM, D), lambda i: (i, 0)),
        ],
        out_shape=[
            jax.ShapeDtypeStruct((N, D), jnp.float32),
            jax.ShapeDtypeStruct((N, D), jnp.float32),
        ],
    )(x, wl, wr, b.reshape(1, D))


def _tc2(acc, cnt, z, wl, wr, b):
    grid = (N // _BM,)
    return pl.pallas_call(
        _tc2_body,
        grid=grid,
        in_specs=[
            pl.BlockSpec((NC, _BM, D), lambda i: (0, i, 0)),
            pl.BlockSpec((NC, _BM, CNT_W), lambda i: (0, i, 0)),
            pl.BlockSpec((_BM, D), lambda i: (i, 0)),
            pl.BlockSpec((D, D), lambda i: (0, 0)),
            pl.BlockSpec((D, D), lambda i: (0, 0)),
            pl.BlockSpec((1, D), lambda i: (0, 0)),
        ],
        out_specs=[
            pl.BlockSpec((_BM, D), lambda i: (i, 0)),
            pl.BlockSpec((_BM, D), lambda i: (i, 0)),
        ],
        out_shape=[
            jax.ShapeDtypeStruct((N, D), jnp.float32),
            jax.ShapeDtypeStruct((N, D), jnp.float32),
        ],
    )(acc, cnt, z, wl, wr, b.reshape(1, D))


def _tc3(acc, cnt, z):
    grid = (N // _BM,)
    return pl.pallas_call(
        _tc3_body,
        grid=grid,
        in_specs=[
            pl.BlockSpec((NC, _BM, D), lambda i: (0, i, 0)),
            pl.BlockSpec((NC, _BM, CNT_W), lambda i: (0, i, 0)),
            pl.BlockSpec((_BM, D), lambda i: (i, 0)),
        ],
        out_specs=pl.BlockSpec((_BM, D), lambda i: (i, 0)),
        out_shape=jax.ShapeDtypeStruct((N, D), jnp.float32),
    )(acc, cnt, z)


def _zero_fill(ref, rows, cols):
    z16 = jnp.zeros((16,), jnp.float32)
    for r in range(rows):
        for c in range(cols // 16):
            ref[r, pl.ds(c * 16, 16)] = z16


def _sc_agg_body(with_cnt, y_hbm, src_hbm, dst_hbm, *rest):
    if with_cnt:
        (acc_out, cnt_out, acc_sh, cnt_sh, src_v, dst_v, dstc_v, rows_v,
         ones_v, zb_v, zbc_v, sem) = rest
    else:
        (acc_out, acc_sh, src_v, dst_v, dstc_v, rows_v,
         zb_v, sem) = rest
        cnt_out = cnt_sh = ones_v = zbc_v = None

    cid = lax.axis_index("c")
    sid = lax.axis_index("s")
    wid = cid * NS + sid
    base_n = sid * ROWS_PER_TILE

    _zero_fill(zb_v, ZB, D)
    if with_cnt:
        _zero_fill(zbc_v, 125, CNT_W)
        o16 = jnp.ones((16,), jnp.float32)
        for r in range(CHUNK):
            ones_v[r, pl.ds(0, 16)] = o16

    for k in range(ROWS_PER_TILE // ZB):
        pltpu.sync_copy(zb_v, acc_sh.at[pl.ds(base_n + k * ZB, ZB)])
    if with_cnt:
        for k in range(ROWS_PER_TILE // 125):
            pltpu.sync_copy(zbc_v, cnt_sh.at[pl.ds(base_n + k * 125, 125)])

    pltpu.sync_copy(src_hbm.at[wid], src_v)
    pltpu.sync_copy(dst_hbm.at[wid], dst_v)

    plsc.subcore_barrier()

    def step(j, carry):
        pltpu.sync_copy(dst_v.at[j], dstc_v)
        pltpu.async_copy(y_hbm.at[src_v.at[j]], rows_v, sem).wait()
        pltpu.sync_copy(rows_v, acc_sh.at[dstc_v], add=True)
        if with_cnt:
            pltpu.sync_copy(ones_v, cnt_sh.at[dstc_v], add=True)
        return carry

    lax.fori_loop(0, N_CHUNKS, step, 0)

    plsc.subcore_barrier()

    pltpu.sync_copy(acc_sh.at[pl.ds(base_n, ROWS_PER_TILE)],
                    acc_out.at[cid, pl.ds(base_n, ROWS_PER_TILE)])
    if with_cnt:
        pltpu.sync_copy(cnt_sh.at[pl.ds(base_n, ROWS_PER_TILE)],
                        cnt_out.at[cid, pl.ds(base_n, ROWS_PER_TILE)])


def _sc_agg(y, src3, dst3, with_cnt):
    mesh = plsc.VectorSubcoreMesh(core_axis_name="c", subcore_axis_name="s")
    out_type = [jax.ShapeDtypeStruct((NC, N, D), jnp.float32)]
    if with_cnt:
        out_type.append(jax.ShapeDtypeStruct((NC, N, CNT_W), jnp.float32))
        scratch = [
            pltpu.VMEM_SHARED((N, D), jnp.float32),
            pltpu.VMEM_SHARED((N, CNT_W), jnp.float32),
            pltpu.VMEM((N_CHUNKS, CHUNK), jnp.int32),
            pltpu.VMEM((N_CHUNKS, CHUNK), jnp.int32),
            pltpu.VMEM((CHUNK,), jnp.int32),
            pltpu.VMEM((CHUNK, D), jnp.float32),
            pltpu.VMEM((CHUNK, CNT_W), jnp.float32),
            pltpu.VMEM((ZB, D), jnp.float32),
            pltpu.VMEM((125, CNT_W), jnp.float32),
            pltpu.SemaphoreType.DMA,
        ]
    else:
        scratch = [
            pltpu.VMEM_SHARED((N, D), jnp.float32),
            pltpu.VMEM((N_CHUNKS, CHUNK), jnp.int32),
            pltpu.VMEM((N_CHUNKS, CHUNK), jnp.int32),
            pltpu.VMEM((CHUNK,), jnp.int32),
            pltpu.VMEM((CHUNK, D), jnp.float32),
            pltpu.VMEM((ZB, D), jnp.float32),
            pltpu.SemaphoreType.DMA,
        ]
    fn = pl.kernel(
        functools.partial(_sc_agg_body, with_cnt),
        out_type=tuple(out_type),
        mesh=mesh,
        scratch_types=scratch,
    )
    return fn(y, src3, dst3)


def kernel(x, edge_index, W1l, W1r, b1, W2l, W2r, b2):
    src = edge_index[0].astype(jnp.int32).reshape(NW, N_CHUNKS, CHUNK)
    dst = edge_index[1].astype(jnp.int32).reshape(NW, N_CHUNKS, CHUNK)

    y1, z1 = _tc1(x, W1l, W1r, b1)
    acc1, cnt = _sc_agg(y1, src, dst, with_cnt=True)
    y2, z2 = _tc2(acc1, cnt, z1, W2l, W2r, b2)
    acc2 = _sc_agg(y2, src, dst, with_cnt=False)
    return _tc3(acc2, cnt, z2)


# speedup vs baseline: 4.5311x; 4.5311x over previous
import functools

import jax
import jax.numpy as jnp
from jax import lax
from jax.experimental import pallas as pl
from jax.experimental.pallas import tpu as pltpu
from jax.experimental.pallas import tpu_sc as plsc

N = 10000
NP = 10240
E = 320000
D = 128

NC = 2
NS = 16
NW = NC * NS
CHUNK = 80
E_PER_W = E // NW
N_CHUNKS = E_PER_W // CHUNK
ROWS_PER_TILE = NP // NS
ZB = 40


def _mm_xt(a, w):
    return lax.dot_general(a, w, (((1,), (1,)), ((), ())),
                           preferred_element_type=jnp.float32)


def _tc1_body(x_ref, wl_ref, wr_ref, b_ref, y_ref, z_ref):
    xb = x_ref[...]
    y_ref[...] = _mm_xt(xb, wl_ref[...])
    z_ref[...] = _mm_xt(xb, wr_ref[...]) + b_ref[...]


def _tc2_body(acc_ref, cnt_ref, z_ref, wl_ref, wr_ref, b_ref, y_ref, z2_ref):
    a = acc_ref[0, :, :] + acc_ref[1, :, :]
    c = cnt_ref[0, :, 0:1] + cnt_ref[1, :, 0:1]
    inv = 1.0 / jnp.maximum(c, 1.0)
    h = jnp.maximum(a * inv + z_ref[...], 0.0)
    y_ref[...] = _mm_xt(h, wl_ref[...])
    z2_ref[...] = _mm_xt(h, wr_ref[...]) + b_ref[...]


def _tc3_body(acc_ref, cnt_ref, z_ref, out_ref):
    a = acc_ref[0, :, :] + acc_ref[1, :, :]
    c = cnt_ref[0, :, 0:1] + cnt_ref[1, :, 0:1]
    inv = 1.0 / jnp.maximum(c, 1.0)
    out_ref[...] = a * inv + z_ref[...]


_BM = 1024


def _tc1(x, wl, wr, b):
    grid = (NP // _BM,)
    return pl.pallas_call(
        _tc1_body,
        grid=grid,
        in_specs=[
            pl.BlockSpec((_BM, D), lambda i: (i, 0)),
            pl.BlockSpec((D, D), lambda i: (0, 0)),
            pl.BlockSpec((D, D), lambda i: (0, 0)),
            pl.BlockSpec((1, D), lambda i: (0, 0)),
        ],
        out_specs=[
            pl.BlockSpec((_BM, D), lambda i: (i, 0)),
            pl.BlockSpec((_BM, D), lambda i: (i, 0)),
        ],
        out_shape=[
            jax.ShapeDtypeStruct((NP, D), jnp.float32),
            jax.ShapeDtypeStruct((NP, D), jnp.float32),
        ],
    )(x, wl, wr, b.reshape(1, D))


def _tc2(acc, cnt, z, wl, wr, b):
    grid = (NP // _BM,)
    return pl.pallas_call(
        _tc2_body,
        grid=grid,
        in_specs=[
            pl.BlockSpec((NC, _BM, D), lambda i: (0, i, 0)),
            pl.BlockSpec((NC, _BM, D), lambda i: (0, i, 0)),
            pl.BlockSpec((_BM, D), lambda i: (i, 0)),
            pl.BlockSpec((D, D), lambda i: (0, 0)),
            pl.BlockSpec((D, D), lambda i: (0, 0)),
            pl.BlockSpec((1, D), lambda i: (0, 0)),
        ],
        out_specs=[
            pl.BlockSpec((_BM, D), lambda i: (i, 0)),
            pl.BlockSpec((_BM, D), lambda i: (i, 0)),
        ],
        out_shape=[
            jax.ShapeDtypeStruct((NP, D), jnp.float32),
            jax.ShapeDtypeStruct((NP, D), jnp.float32),
        ],
    )(acc, cnt, z, wl, wr, b.reshape(1, D))


def _tc3(acc, cnt, z):
    grid = (NP // _BM,)
    return pl.pallas_call(
        _tc3_body,
        grid=grid,
        in_specs=[
            pl.BlockSpec((NC, _BM, D), lambda i: (0, i, 0)),
            pl.BlockSpec((NC, _BM, D), lambda i: (0, i, 0)),
            pl.BlockSpec((_BM, D), lambda i: (i, 0)),
        ],
        out_specs=pl.BlockSpec((_BM, D), lambda i: (i, 0)),
        out_shape=jax.ShapeDtypeStruct((NP, D), jnp.float32),
    )(acc, cnt, z)


def _zero_fill(ref, rows, cols):
    z16 = jnp.zeros((16,), jnp.float32)
    for r in range(rows):
        for c in range(cols // 16):
            ref[r, pl.ds(c * 16, 16)] = z16


def _sc_agg_body(y_hbm, src_hbm, dst_hbm, acc_out, acc_sh, srcc_v, dstc_v,
                 rows_v, zb_v, sem):
    cid = lax.axis_index("c")
    sid = lax.axis_index("s")
    wid = cid * NS + sid
    base_n = sid * ROWS_PER_TILE

    _zero_fill(zb_v, ZB, D)
    for k in range(ROWS_PER_TILE // ZB):
        pltpu.sync_copy(zb_v, acc_sh.at[pl.ds(base_n + k * ZB, ZB)])

    plsc.subcore_barrier()

    def step(j, carry):
        pltpu.sync_copy(src_hbm.at[wid, j], srcc_v)
        pltpu.sync_copy(dst_hbm.at[wid, j], dstc_v)
        pltpu.async_copy(y_hbm.at[srcc_v], rows_v, sem).wait()
        pltpu.sync_copy(rows_v, acc_sh.at[dstc_v], add=True)
        return carry

    lax.fori_loop(0, N_CHUNKS, step, 0)

    plsc.subcore_barrier()

    pltpu.sync_copy(acc_sh.at[pl.ds(base_n, ROWS_PER_TILE)],
                    acc_out.at[cid, pl.ds(base_n, ROWS_PER_TILE)])


def _sc_cnt_body(dst_hbm, cnt_out, cnt_sh, dstc_v, ones_v, zb_v, sem):
    cid = lax.axis_index("c")
    sid = lax.axis_index("s")
    wid = cid * NS + sid
    base_n = sid * ROWS_PER_TILE

    _zero_fill(zb_v, ZB, D)
    o16 = jnp.ones((16,), jnp.float32)
    for r in range(CHUNK):
        for c in range(D // 16):
            ones_v[r, pl.ds(c * 16, 16)] = o16
    for k in range(ROWS_PER_TILE // ZB):
        pltpu.sync_copy(zb_v, cnt_sh.at[pl.ds(base_n + k * ZB, ZB)])

    plsc.subcore_barrier()

    def step(j, carry):
        pltpu.sync_copy(dst_hbm.at[wid, j], dstc_v)
        pltpu.sync_copy(ones_v, cnt_sh.at[dstc_v], add=True)
        return carry

    lax.fori_loop(0, N_CHUNKS, step, 0)

    plsc.subcore_barrier()
    pltpu.sync_copy(cnt_sh.at[pl.ds(base_n, ROWS_PER_TILE)],
                    cnt_out.at[cid, pl.ds(base_n, ROWS_PER_TILE)])


def _sc_agg(y, src3, dst3):
    mesh = plsc.VectorSubcoreMesh(core_axis_name="c", subcore_axis_name="s")
    fn = pl.kernel(
        _sc_agg_body,
        out_type=jax.ShapeDtypeStruct((NC, NP, D), jnp.float32),
        mesh=mesh,
        scratch_types=[
            pltpu.VMEM_SHARED((NP, D), jnp.float32),
            pltpu.VMEM((CHUNK,), jnp.int32),
            pltpu.VMEM((CHUNK,), jnp.int32),
            pltpu.VMEM((CHUNK, D), jnp.float32),
            pltpu.VMEM((ZB, D), jnp.float32),
            pltpu.SemaphoreType.DMA,
        ],
    )
    return fn(y, src3, dst3)


def _sc_cnt(dst3):
    mesh = plsc.VectorSubcoreMesh(core_axis_name="c", subcore_axis_name="s")
    fn = pl.kernel(
        _sc_cnt_body,
        out_type=jax.ShapeDtypeStruct((NC, NP, D), jnp.float32),
        mesh=mesh,
        scratch_types=[
            pltpu.VMEM_SHARED((NP, D), jnp.float32),
            pltpu.VMEM((CHUNK,), jnp.int32),
            pltpu.VMEM((CHUNK, D), jnp.float32),
            pltpu.VMEM((ZB, D), jnp.float32),
            pltpu.SemaphoreType.DMA,
        ],
    )
    return fn(dst3)


def kernel(x, edge_index, W1l, W1r, b1, W2l, W2r, b2):
    src = edge_index[0].astype(jnp.int32).reshape(NW, N_CHUNKS, CHUNK)
    dst = edge_index[1].astype(jnp.int32).reshape(NW, N_CHUNKS, CHUNK)
    xp = jnp.pad(x, ((0, NP - N), (0, 0)))

    cnt = _sc_cnt(dst)
    y1, z1 = _tc1(xp, W1l, W1r, b1)
    acc1 = _sc_agg(y1, src, dst)
    y2, z2 = _tc2(acc1, cnt, z1, W2l, W2r, b2)
    acc2 = _sc_agg(y2, src, dst)
    return _tc3(acc2, cnt, z2)[:N]


# speedup vs baseline: 5.7900x; 1.2778x over previous
import functools

import jax
import jax.numpy as jnp
from jax import lax
from jax.experimental import pallas as pl
from jax.experimental.pallas import tpu as pltpu
from jax.experimental.pallas import tpu_sc as plsc

N = 10000
NP = 10240
E = 320000
D = 128

NC = 2
NS = 16
NW = NC * NS
CHUNK = 80
E_PER_W = E // NW
N_CHUNKS = E_PER_W // CHUNK
ROWS_PER_TILE = NP // NS
ZB = 40


def _mm_xt(a, w):
    return lax.dot_general(a, w, (((1,), (1,)), ((), ())),
                           preferred_element_type=jnp.float32)


def _tc1_body(x_ref, wl_ref, wr_ref, b_ref, y_ref, z_ref):
    xb = x_ref[...]
    y_ref[...] = _mm_xt(xb, wl_ref[...])
    z_ref[...] = _mm_xt(xb, wr_ref[...]) + b_ref[...]


def _tc2_body(acc_ref, cnt_ref, z_ref, wl_ref, wr_ref, b_ref, y_ref, z2_ref):
    a = acc_ref[0, :, :] + acc_ref[1, :, :]
    c = cnt_ref[0, :, 0:1] + cnt_ref[1, :, 0:1]
    inv = 1.0 / jnp.maximum(c, 1.0)
    h = jnp.maximum(a * inv + z_ref[...], 0.0)
    y_ref[...] = _mm_xt(h, wl_ref[...])
    z2_ref[...] = _mm_xt(h, wr_ref[...]) + b_ref[...]


def _tc3_body(acc_ref, cnt_ref, z_ref, out_ref):
    a = acc_ref[0, :, :] + acc_ref[1, :, :]
    c = cnt_ref[0, :, 0:1] + cnt_ref[1, :, 0:1]
    inv = 1.0 / jnp.maximum(c, 1.0)
    out_ref[...] = a * inv + z_ref[...]


_BM = 1024


def _tc1(x, wl, wr, b):
    grid = (NP // _BM,)
    return pl.pallas_call(
        _tc1_body,
        grid=grid,
        in_specs=[
            pl.BlockSpec((_BM, D), lambda i: (i, 0)),
            pl.BlockSpec((D, D), lambda i: (0, 0)),
            pl.BlockSpec((D, D), lambda i: (0, 0)),
            pl.BlockSpec((1, D), lambda i: (0, 0)),
        ],
        out_specs=[
            pl.BlockSpec((_BM, D), lambda i: (i, 0)),
            pl.BlockSpec((_BM, D), lambda i: (i, 0)),
        ],
        out_shape=[
            jax.ShapeDtypeStruct((NP, D), jnp.float32),
            jax.ShapeDtypeStruct((NP, D), jnp.float32),
        ],
    )(x, wl, wr, b.reshape(1, D))


def _tc2(acc, cnt, z, wl, wr, b):
    grid = (NP // _BM,)
    return pl.pallas_call(
        _tc2_body,
        grid=grid,
        in_specs=[
            pl.BlockSpec((NC, _BM, D), lambda i: (0, i, 0)),
            pl.BlockSpec((NC, _BM, D), lambda i: (0, i, 0)),
            pl.BlockSpec((_BM, D), lambda i: (i, 0)),
            pl.BlockSpec((D, D), lambda i: (0, 0)),
            pl.BlockSpec((D, D), lambda i: (0, 0)),
            pl.BlockSpec((1, D), lambda i: (0, 0)),
        ],
        out_specs=[
            pl.BlockSpec((_BM, D), lambda i: (i, 0)),
            pl.BlockSpec((_BM, D), lambda i: (i, 0)),
        ],
        out_shape=[
            jax.ShapeDtypeStruct((NP, D), jnp.float32),
            jax.ShapeDtypeStruct((NP, D), jnp.float32),
        ],
    )(acc, cnt, z, wl, wr, b.reshape(1, D))


def _tc3(acc, cnt, z):
    grid = (NP // _BM,)
    return pl.pallas_call(
        _tc3_body,
        grid=grid,
        in_specs=[
            pl.BlockSpec((NC, _BM, D), lambda i: (0, i, 0)),
            pl.BlockSpec((NC, _BM, D), lambda i: (0, i, 0)),
            pl.BlockSpec((_BM, D), lambda i: (i, 0)),
        ],
        out_specs=pl.BlockSpec((_BM, D), lambda i: (i, 0)),
        out_shape=jax.ShapeDtypeStruct((NP, D), jnp.float32),
    )(acc, cnt, z)


def _zero_fill(ref, rows, cols):
    z16 = jnp.zeros((16,), jnp.float32)
    for r in range(rows):
        for c in range(cols // 16):
            ref[r, pl.ds(c * 16, 16)] = z16


def _zero_fill3(ref, b, rows, cols):
    z16 = jnp.zeros((16,), jnp.float32)
    for r in range(rows):
        for c in range(cols // 16):
            ref[b, r, pl.ds(c * 16, 16)] = z16


def _sc_agg_body(y_hbm, src_hbm, dst_hbm, acc_out, acc_sh, sA, dA, sB, dB,
                 rows_v, semA, semB):
    cid = lax.axis_index("c")
    sid = lax.axis_index("s")
    wid = cid * NS + sid
    base_n = sid * ROWS_PER_TILE

    _zero_fill3(rows_v, 0, CHUNK, D)
    for k in range(ROWS_PER_TILE // CHUNK):
        pltpu.sync_copy(rows_v.at[0], acc_sh.at[pl.ds(base_n + k * CHUNK, CHUNK)])

    plsc.subcore_barrier()

    pltpu.sync_copy(src_hbm.at[wid, 0], sA)
    pltpu.sync_copy(dst_hbm.at[wid, 0], dA)
    g0 = pltpu.async_copy(y_hbm.at[sA], rows_v.at[0], semA)
    g0.wait()

    def step(j, carry):
        c0 = 2 * j + 1
        c1 = 2 * j + 2
        pltpu.sync_copy(src_hbm.at[wid, c0], sB)
        pltpu.sync_copy(dst_hbm.at[wid, c0], dB)
        gB = pltpu.async_copy(y_hbm.at[sB], rows_v.at[1], semB)
        pltpu.sync_copy(rows_v.at[0], acc_sh.at[dA], add=True)
        pltpu.sync_copy(src_hbm.at[wid, c1], sA)
        pltpu.sync_copy(dst_hbm.at[wid, c1], dA)
        gA = pltpu.async_copy(y_hbm.at[sA], rows_v.at[0], semA)
        gB.wait()
        pltpu.sync_copy(rows_v.at[1], acc_sh.at[dB], add=True)
        gA.wait()
        return carry

    lax.fori_loop(0, (N_CHUNKS - 1) // 2, step, 0)
    pltpu.sync_copy(rows_v.at[0], acc_sh.at[dA], add=True)

    plsc.subcore_barrier()

    pltpu.sync_copy(acc_sh.at[pl.ds(base_n, ROWS_PER_TILE)],
                    acc_out.at[cid, pl.ds(base_n, ROWS_PER_TILE)])


def _sc_cnt_body(dst_hbm, cnt_out, cnt_sh, dstc_v, ones_v, zb_v, sem):
    cid = lax.axis_index("c")
    sid = lax.axis_index("s")
    wid = cid * NS + sid
    base_n = sid * ROWS_PER_TILE

    _zero_fill(zb_v, ZB, D)
    o16 = jnp.ones((16,), jnp.float32)
    for r in range(CHUNK):
        for c in range(D // 16):
            ones_v[r, pl.ds(c * 16, 16)] = o16
    for k in range(ROWS_PER_TILE // ZB):
        pltpu.sync_copy(zb_v, cnt_sh.at[pl.ds(base_n + k * ZB, ZB)])

    plsc.subcore_barrier()

    def step(j, carry):
        pltpu.sync_copy(dst_hbm.at[wid, j], dstc_v)
        pltpu.sync_copy(ones_v, cnt_sh.at[dstc_v], add=True)
        return carry

    lax.fori_loop(0, N_CHUNKS, step, 0)

    plsc.subcore_barrier()
    pltpu.sync_copy(cnt_sh.at[pl.ds(base_n, ROWS_PER_TILE)],
                    cnt_out.at[cid, pl.ds(base_n, ROWS_PER_TILE)])


def _sc_agg(y, src3, dst3):
    mesh = plsc.VectorSubcoreMesh(core_axis_name="c", subcore_axis_name="s")
    fn = pl.kernel(
        _sc_agg_body,
        out_type=jax.ShapeDtypeStruct((NC, NP, D), jnp.float32),
        mesh=mesh,
        scratch_types=[
            pltpu.VMEM_SHARED((NP, D), jnp.float32),
            pltpu.VMEM((CHUNK,), jnp.int32),
            pltpu.VMEM((CHUNK,), jnp.int32),
            pltpu.VMEM((CHUNK,), jnp.int32),
            pltpu.VMEM((CHUNK,), jnp.int32),
            pltpu.VMEM((2, CHUNK, D), jnp.float32),
            pltpu.SemaphoreType.DMA,
            pltpu.SemaphoreType.DMA,
        ],
    )
    return fn(y, src3, dst3)


def _sc_cnt(dst3):
    mesh = plsc.VectorSubcoreMesh(core_axis_name="c", subcore_axis_name="s")
    fn = pl.kernel(
        _sc_cnt_body,
        out_type=jax.ShapeDtypeStruct((NC, NP, D), jnp.float32),
        mesh=mesh,
        scratch_types=[
            pltpu.VMEM_SHARED((NP, D), jnp.float32),
            pltpu.VMEM((CHUNK,), jnp.int32),
            pltpu.VMEM((CHUNK, D), jnp.float32),
            pltpu.VMEM((ZB, D), jnp.float32),
            pltpu.SemaphoreType.DMA,
        ],
    )
    return fn(dst3)


def kernel(x, edge_index, W1l, W1r, b1, W2l, W2r, b2):
    src = edge_index[0].astype(jnp.int32).reshape(NW, N_CHUNKS, CHUNK)
    dst = edge_index[1].astype(jnp.int32).reshape(NW, N_CHUNKS, CHUNK)
    xp = jnp.pad(x, ((0, NP - N), (0, 0)))

    cnt = _sc_cnt(dst)
    y1, z1 = _tc1(xp, W1l, W1r, b1)
    acc1 = _sc_agg(y1, src, dst)
    y2, z2 = _tc2(acc1, cnt, z1, W2l, W2r, b2)
    acc2 = _sc_agg(y2, src, dst)
    return _tc3(acc2, cnt, z2)[:N]
